# Initial kernel scaffold; baseline (speedup 1.0000x reference)
import jax
import jax.numpy as jnp
from jax.experimental import pallas as pl


def kernel(hidden_states, weight):
    raise NotImplementedError("write your pallas kernel here")


# speedup vs baseline: 1.3407x; 1.3407x over previous
import jax
import jax.numpy as jnp
from jax.experimental import pallas as pl
from jax.experimental.pallas import tpu as pltpu

E = 64
K = 8
G = 8
KG = 3
GS = E // G


def _gate_kernel(h_ref, w_ref, idx_ref, wgt_ref):
    h = h_ref[...]
    w = w_ref[...]
    logits = jax.lax.dot_general(
        h, w, (((1,), (1,)), ((), ())),
        preferred_element_type=jnp.float32)
    m = jnp.max(logits, axis=-1, keepdims=True)
    e = jnp.exp(logits - m)

    ge = jnp.concatenate(
        [jnp.max(e[:, g * GS:(g + 1) * GS], axis=-1, keepdims=True)
         for g in range(G)], axis=-1)

    gcols = jax.lax.broadcasted_iota(jnp.int32, ge.shape, 1)
    gcur = ge
    gsel = jnp.zeros_like(ge)
    for _ in range(KG):
        gmv = jnp.max(gcur, axis=-1, keepdims=True)
        gamax = jnp.min(jnp.where(gcur == gmv, gcols, G), axis=-1, keepdims=True)
        hit = gcols == gamax
        gsel = jnp.where(hit, 1.0, gsel)
        gcur = jnp.where(hit, -1.0, gcur)

    onehot = (jax.lax.broadcasted_iota(jnp.int32, (G, E), 1) // GS ==
              jax.lax.broadcasted_iota(jnp.int32, (G, E), 0)).astype(jnp.float32)
    emask = jax.lax.dot_general(
        gsel, onehot, (((1,), (0,)), ((), ())),
        preferred_element_type=jnp.float32)
    cur = e * emask

    cols = jax.lax.broadcasted_iota(jnp.int32, cur.shape, 1)
    idxs, vals = [], []
    for _ in range(K):
        mv = jnp.max(cur, axis=-1, keepdims=True)
        is_max = cur == mv
        amax = jnp.min(jnp.where(is_max, cols, E), axis=-1, keepdims=True)
        idxs.append(amax)
        vals.append(mv)
        cur = jnp.where(cols == amax, -1.0, cur)
    vals = jnp.concatenate(vals, axis=-1)
    idxs = jnp.concatenate(idxs, axis=-1)
    denom = jnp.sum(vals, axis=-1, keepdims=True)
    idx_ref[...] = idxs
    wgt_ref[...] = vals / denom


def kernel(hidden_states, weight):
    T, D = hidden_states.shape
    B = 256
    grid = (T // B,)
    idx, wgt = pl.pallas_call(
        _gate_kernel,
        grid=grid,
        in_specs=[
            pl.BlockSpec((B, D), lambda i: (i, 0)),
            pl.BlockSpec((E, D), lambda i: (0, 0)),
        ],
        out_specs=[
            pl.BlockSpec((B, K), lambda i: (i, 0)),
            pl.BlockSpec((B, K), lambda i: (i, 0)),
        ],
        out_shape=[
            jax.ShapeDtypeStruct((T, K), jnp.int32),
            jax.ShapeDtypeStruct((T, K), jnp.float32),
        ],
    )(hidden_states, weight)
    return idx, wgt


# speedup vs baseline: 2.9782x; 2.2214x over previous
import jax
import jax.numpy as jnp
from jax.experimental import pallas as pl
from jax.experimental.pallas import tpu as pltpu

E = 64
K = 8
G = 8
KG = 3
GS = E // G


def _gate_kernel(h_ref, w_ref, idx_ref, wgt_ref):
    h = h_ref[...]
    w = w_ref[...]
    logits = jax.lax.dot_general(
        h, w, (((1,), (1,)), ((), ())),
        preferred_element_type=jnp.float32)
    m = jnp.max(logits, axis=-1, keepdims=True)
    e = jnp.exp(logits - m)

    ge = jnp.concatenate(
        [jnp.max(e[:, g * GS:(g + 1) * GS], axis=-1, keepdims=True)
         for g in range(G)], axis=-1)

    gcols = jax.lax.broadcasted_iota(jnp.int32, ge.shape, 1).astype(jnp.float32)
    gcur = ge
    gsel = jnp.zeros_like(ge)
    for _ in range(KG):
        gmv = jnp.max(gcur, axis=-1, keepdims=True)
        gamax = jnp.min(jnp.where(gcur == gmv, gcols, float(G)),
                        axis=-1, keepdims=True)
        hit = gcols == gamax
        gsel = jnp.where(hit, 1.0, gsel)
        gcur = jnp.where(hit, -1.0, gcur)

    onehot = (jax.lax.broadcasted_iota(jnp.int32, (G, E), 1) // GS ==
              jax.lax.broadcasted_iota(jnp.int32, (G, E), 0)).astype(jnp.float32)
    emask = jax.lax.dot_general(
        gsel, onehot, (((1,), (0,)), ((), ())),
        preferred_element_type=jnp.float32)
    cur = e * emask

    cols = jax.lax.broadcasted_iota(jnp.int32, cur.shape, 1).astype(jnp.float32)
    idxs, vals = [], []
    for _ in range(K):
        mv = jnp.max(cur, axis=-1, keepdims=True)
        amax = jnp.min(jnp.where(cur == mv, cols, float(E)),
                       axis=-1, keepdims=True)
        idxs.append(amax)
        vals.append(mv)
        cur = jnp.where(cols == amax, -1.0, cur)
    vals = jnp.concatenate(vals, axis=-1)
    idxs_f = jnp.concatenate(idxs, axis=-1)
    denom = jnp.sum(vals, axis=-1, keepdims=True)
    idx_ref[...] = idxs_f.astype(jnp.int32)
    wgt_ref[...] = vals / denom


def kernel(hidden_states, weight):
    T, D = hidden_states.shape
    B = 1024
    grid = (T // B,)
    idx, wgt = pl.pallas_call(
        _gate_kernel,
        grid=grid,
        in_specs=[
            pl.BlockSpec((B, D), lambda i: (i, 0)),
            pl.BlockSpec((E, D), lambda i: (0, 0)),
        ],
        out_specs=[
            pl.BlockSpec((B, K), lambda i: (i, 0)),
            pl.BlockSpec((B, K), lambda i: (i, 0)),
        ],
        out_shape=[
            jax.ShapeDtypeStruct((T, K), jnp.int32),
            jax.ShapeDtypeStruct((T, K), jnp.float32),
        ],
    )(hidden_states, weight)
    return idx, wgt


# speedup vs baseline: 4.8501x; 1.6285x over previous
import jax
import jax.numpy as jnp
from jax.experimental import pallas as pl
from jax.experimental.pallas import tpu as pltpu

E = 64
K = 8
G = 8
KG = 3
GS = E // G
NEG = -3.0e38


def _gate_kernel(h_ref, w_ref, idx_ref, wgt_ref):
    h = h_ref[...]
    w = w_ref[...]
    logits = jax.lax.dot_general(
        w, h, (((1,), (1,)), ((), ())),
        preferred_element_type=jnp.float32)
    B = logits.shape[1]

    ge = jnp.max(logits.reshape(G, GS, B), axis=1)

    grows = jax.lax.broadcasted_iota(jnp.int32, ge.shape, 0).astype(jnp.float32)
    gsel = jnp.zeros_like(ge)
    for _ in range(KG):
        gmv = jnp.max(ge, axis=0, keepdims=True)
        gamax = jnp.min(jnp.where(ge == gmv, grows, float(G)),
                        axis=0, keepdims=True)
        hit = grows == gamax
        gsel = jnp.where(hit, 1.0, gsel)
        ge = jnp.where(hit, NEG, ge)

    onehot = (jax.lax.broadcasted_iota(jnp.int32, (E, G), 0) // GS ==
              jax.lax.broadcasted_iota(jnp.int32, (E, G), 1)).astype(jnp.float32)
    emask = jax.lax.dot_general(
        onehot, gsel, (((1,), (0,)), ((), ())),
        preferred_element_type=jnp.float32)
    cur = jnp.where(emask == 1.0, logits, NEG)

    rows = jax.lax.broadcasted_iota(jnp.int32, cur.shape, 0).astype(jnp.float32)
    idxs, vals = [], []
    for _ in range(K):
        mv = jnp.max(cur, axis=0, keepdims=True)
        amax = jnp.min(jnp.where(cur == mv, rows, float(E)),
                       axis=0, keepdims=True)
        idxs.append(amax)
        vals.append(mv)
        cur = jnp.where(rows == amax, NEG, cur)
    vals = jnp.concatenate(vals, axis=0)
    idxs_f = jnp.concatenate(idxs, axis=0)
    ev = jnp.exp(vals - vals[0:1, :])
    denom = jnp.sum(ev, axis=0, keepdims=True)
    wgt = ev / denom
    idx_ref[...] = idxs_f.T.astype(jnp.int32)
    wgt_ref[...] = wgt.T


def kernel(hidden_states, weight):
    T, D = hidden_states.shape
    B = 1024
    grid = (T // B,)
    idx, wgt = pl.pallas_call(
        _gate_kernel,
        grid=grid,
        in_specs=[
            pl.BlockSpec((B, D), lambda i: (i, 0)),
            pl.BlockSpec((E, D), lambda i: (0, 0)),
        ],
        out_specs=[
            pl.BlockSpec((B, K), lambda i: (i, 0)),
            pl.BlockSpec((B, K), lambda i: (i, 0)),
        ],
        out_shape=[
            jax.ShapeDtypeStruct((T, K), jnp.int32),
            jax.ShapeDtypeStruct((T, K), jnp.float32),
        ],
    )(hidden_states, weight)
    return idx, wgt
